# baseline (speedup 1.0000x reference)
import jax
import jax.numpy as jnp
from jax.experimental import pallas as pl
from jax.experimental.pallas import tpu as pltpu

MAX_B = 8
MAX_S = 2048
N_HEADS = 16
HEAD_DIM = 128
Q_LEN = 16


def _body(pos_ref, kc, vc, kv, vv, ko, vo, sem, row_sem):
    for b in range(MAX_B):
        pltpu.make_async_copy(kc.at[b], ko.at[b], sem).start()
        pltpu.make_async_copy(vc.at[b], vo.at[b], sem).start()
    for b in range(MAX_B):
        pltpu.make_async_copy(kc.at[b], ko.at[b], sem).wait()
        pltpu.make_async_copy(vc.at[b], vo.at[b], sem).wait()
    for i in range(Q_LEN):
        p = pos_ref[i]
        pltpu.make_async_copy(
            kv.at[:, :, pl.ds(i, 1), :], ko.at[:, :, pl.ds(p, 1), :], row_sem
        ).start()
        pltpu.make_async_copy(
            vv.at[:, :, pl.ds(i, 1), :], vo.at[:, :, pl.ds(p, 1), :], row_sem
        ).start()
    for i in range(Q_LEN):
        p = pos_ref[i]
        pltpu.make_async_copy(
            kv.at[:, :, pl.ds(i, 1), :], ko.at[:, :, pl.ds(p, 1), :], row_sem
        ).wait()
        pltpu.make_async_copy(
            vv.at[:, :, pl.ds(i, 1), :], vo.at[:, :, pl.ds(p, 1), :], row_sem
        ).wait()


def kernel(k_cache, v_cache, input_pos, k_val, v_val):
    out_shape = jax.ShapeDtypeStruct((MAX_B, N_HEADS, MAX_S, HEAD_DIM), jnp.float32)
    any_spec = pl.BlockSpec(memory_space=pl.ANY)
    return pl.pallas_call(
        _body,
        in_specs=[
            pl.BlockSpec(memory_space=pltpu.SMEM),
            any_spec, any_spec, any_spec, any_spec,
        ],
        out_specs=[any_spec, any_spec],
        out_shape=[out_shape, out_shape],
        scratch_shapes=[pltpu.SemaphoreType.DMA, pltpu.SemaphoreType.DMA],
    )(input_pos, k_cache, v_cache, k_val, v_val)


# speedup vs baseline: 49.2185x; 49.2185x over previous
import jax
import jax.numpy as jnp
from jax.experimental import pallas as pl
from jax.experimental.pallas import tpu as pltpu

MAX_B = 8
MAX_S = 2048
N_HEADS = 16
HEAD_DIM = 128
Q_LEN = 16

ROWS = MAX_B * N_HEADS * MAX_S
BLK = 16384
SLABS = BLK // MAX_S


def _body(pos_ref, cache_ref, val_ref, out_ref):
    out_ref[...] = cache_ref[...]
    for s in range(SLABS):
        for i in range(Q_LEN):
            p = pos_ref[i]
            out_ref[pl.ds(s * MAX_S + p, 1), :] = val_ref[pl.ds(s * Q_LEN + i, 1), :]


def kernel(k_cache, v_cache, input_pos, k_val, v_val):
    shape4 = (MAX_B, N_HEADS, MAX_S, HEAD_DIM)
    cache_spec = pl.BlockSpec((BLK, HEAD_DIM), lambda i: (i, 0))
    val_spec = pl.BlockSpec((SLABS * Q_LEN, HEAD_DIM), lambda i: (i, 0))
    update = pl.pallas_call(
        _body,
        grid=(ROWS // BLK,),
        in_specs=[
            pl.BlockSpec(memory_space=pltpu.SMEM),
            cache_spec,
            val_spec,
        ],
        out_specs=cache_spec,
        out_shape=jax.ShapeDtypeStruct((ROWS, HEAD_DIM), jnp.float32),
    )
    k_out = update(input_pos, k_cache.reshape(ROWS, HEAD_DIM),
                   k_val.reshape(MAX_B * N_HEADS * Q_LEN, HEAD_DIM))
    v_out = update(input_pos, v_cache.reshape(ROWS, HEAD_DIM),
                   v_val.reshape(MAX_B * N_HEADS * Q_LEN, HEAD_DIM))
    return k_out.reshape(shape4), v_out.reshape(shape4)
